# Initial kernel scaffold; baseline (speedup 1.0000x reference)
import functools

import jax
import jax.numpy as jnp
from jax import lax
from jax.experimental import pallas as pl
from jax.experimental.pallas import tpu as pltpu
from jax.experimental.pallas import tpu_sc as plsc

NC = 2
NS = 16
NW = NC * NS
L = 16

TABLE_PAD = 128

B, N = 16384, 200
TOTAL = B * N
PER_W = TOTAL // NW
BLK = 12800
NBLK = PER_W // BLK


def _sc_body(in_hbm, idx_hbm, sd_hbm, mu_hbm, out_hbm,
             sd_v, mu_v, idx_v, in_v, out_v):
    wid = lax.axis_index("s") * NC + lax.axis_index("c")
    base = wid * PER_W

    pltpu.sync_copy(sd_hbm, sd_v)
    pltpu.sync_copy(mu_hbm, mu_v)

    def blk_body(i, _):
        off = base + i * BLK
        pltpu.sync_copy(idx_hbm.at[pl.ds(off, BLK)], idx_v)
        pltpu.sync_copy(in_hbm.at[pl.ds(off, BLK)], in_v)

        @functools.partial(plsc.parallel_loop, 0, BLK, L, unroll=4)
        def vec_body(s):
            idx = idx_v[pl.ds(s, L)]
            sd = plsc.load_gather(sd_v, [idx])
            mu = plsc.load_gather(mu_v, [idx])
            out_v[pl.ds(s, L)] = in_v[pl.ds(s, L)] * sd + mu

        pltpu.sync_copy(out_v, out_hbm.at[pl.ds(off, BLK)])
        return 0

    lax.fori_loop(0, NBLK, blk_body, 0)


@jax.jit
def _scale_shift(x_flat, z_flat, sd_pad, mu_pad):
    run = pl.kernel(
        _sc_body,
        out_type=jax.ShapeDtypeStruct((TOTAL,), jnp.float32),
        mesh=plsc.VectorSubcoreMesh(core_axis_name="c", subcore_axis_name="s"),
        scratch_types=[
            pltpu.VMEM((TABLE_PAD,), jnp.float32),
            pltpu.VMEM((TABLE_PAD,), jnp.float32),
            pltpu.VMEM((BLK,), jnp.int32),
            pltpu.VMEM((BLK,), jnp.float32),
            pltpu.VMEM((BLK,), jnp.float32),
        ],
    )
    return run(x_flat, z_flat, sd_pad, mu_pad)


def kernel(inputs, atomic_numbers, mean, stddev):
    x_flat = inputs.reshape(TOTAL)
    z_flat = atomic_numbers.reshape(TOTAL).astype(jnp.int32)
    pad = TABLE_PAD - mean.shape[0]
    sd_pad = jnp.pad(stddev.astype(jnp.float32), (0, pad))
    mu_pad = jnp.pad(mean.astype(jnp.float32), (0, pad))
    y = _scale_shift(x_flat, z_flat, sd_pad, mu_pad)
    return y.reshape(inputs.shape)


# speedup vs baseline: 393.6359x; 393.6359x over previous
import jax
import jax.numpy as jnp
from jax import lax
from jax.experimental import pallas as pl
from jax.experimental.pallas import tpu as pltpu
from jax.experimental.pallas import tpu_sc as plsc

NC = 2
NS = 16
NW = NC * NS
L = 16

TABLE_PAD = 128

B, N = 16384, 200
TOTAL = B * N
PER_W = TOTAL // NW
BLK = 12800
NBLK = PER_W // BLK


def _sc_body(in_hbm, idx_hbm, sd_hbm, mu_hbm, out_hbm,
             sd_v, mu_v, idx_v, in_v, out_v):
    wid = lax.axis_index("s") * NC + lax.axis_index("c")
    base = wid * PER_W

    pltpu.sync_copy(sd_hbm, sd_v)
    pltpu.sync_copy(mu_hbm, mu_v)

    def blk_body(i, _):
        off = base + i * BLK
        pltpu.sync_copy(idx_hbm.at[pl.ds(off, BLK)], idx_v)
        pltpu.sync_copy(in_hbm.at[pl.ds(off, BLK)], in_v)

        @plsc.parallel_loop(0, BLK, step=L, unroll=4)
        def vec_body(s):
            idx = idx_v[pl.ds(s, L)]
            sd = plsc.load_gather(sd_v, [idx])
            mu = plsc.load_gather(mu_v, [idx])
            out_v[pl.ds(s, L)] = in_v[pl.ds(s, L)] * sd + mu

        pltpu.sync_copy(out_v, out_hbm.at[pl.ds(off, BLK)])
        return 0

    lax.fori_loop(0, NBLK, blk_body, 0)


@jax.jit
def _scale_shift(x_flat, z_flat, sd_pad, mu_pad):
    run = pl.kernel(
        _sc_body,
        out_type=jax.ShapeDtypeStruct((TOTAL,), jnp.float32),
        mesh=plsc.VectorSubcoreMesh(core_axis_name="c", subcore_axis_name="s"),
        compiler_params=pltpu.CompilerParams(needs_layout_passes=False),
        scratch_types=[
            pltpu.VMEM((TABLE_PAD,), jnp.float32),
            pltpu.VMEM((TABLE_PAD,), jnp.float32),
            pltpu.VMEM((BLK,), jnp.int32),
            pltpu.VMEM((BLK,), jnp.float32),
            pltpu.VMEM((BLK,), jnp.float32),
        ],
    )
    return run(x_flat, z_flat, sd_pad, mu_pad)


def kernel(inputs, atomic_numbers, mean, stddev):
    x_flat = inputs.reshape(TOTAL)
    z_flat = atomic_numbers.reshape(TOTAL).astype(jnp.int32)
    pad = TABLE_PAD - mean.shape[0]
    sd_pad = jnp.pad(stddev.astype(jnp.float32), (0, pad))
    mu_pad = jnp.pad(mean.astype(jnp.float32), (0, pad))
    y = _scale_shift(x_flat, z_flat, sd_pad, mu_pad)
    return y.reshape(inputs.shape)


# speedup vs baseline: 592.7680x; 1.5059x over previous
import jax
import jax.numpy as jnp
from jax import lax
from jax.experimental import pallas as pl
from jax.experimental.pallas import tpu as pltpu
from jax.experimental.pallas import tpu_sc as plsc

NC = 2
NS = 16
NW = NC * NS
L = 16

TABLE_PAD = 128

ROWS, COLS = 16384, 200
RPW = ROWS // NW
RBLK = 64
NBLK = RPW // RBLK
BLKE = RBLK * COLS


def _sc_body(in_hbm, idx_hbm, sd_hbm, mu_hbm, out_hbm,
             sd_v, mu_v, idx_v, in_v, out_v):
    wid = lax.axis_index("s") * NC + lax.axis_index("c")
    r0 = wid * RPW

    pltpu.sync_copy(sd_hbm, sd_v)
    pltpu.sync_copy(mu_hbm, mu_v)

    col_offs = [c * L for c in range(COLS // L)] + [COLS - L]

    def blk_body(i, _):
        rb = r0 + i * RBLK
        pltpu.sync_copy(idx_hbm.at[pl.ds(rb, RBLK), :], idx_v)
        pltpu.sync_copy(in_hbm.at[pl.ds(rb, RBLK), :], in_v)

        @plsc.parallel_loop(0, RBLK, step=1, unroll=2)
        def row_body(r):
            for c in col_offs:
                idx = idx_v[r, pl.ds(c, L)]
                sd = plsc.load_gather(sd_v, [idx])
                mu = plsc.load_gather(mu_v, [idx])
                out_v[r, pl.ds(c, L)] = in_v[r, pl.ds(c, L)] * sd + mu

        pltpu.sync_copy(out_v, out_hbm.at[pl.ds(rb, RBLK), :])
        return 0

    lax.fori_loop(0, NBLK, blk_body, 0)


@jax.jit
def _scale_shift(x, z, sd_pad, mu_pad):
    run = pl.kernel(
        _sc_body,
        out_type=jax.ShapeDtypeStruct((ROWS, COLS), jnp.float32),
        mesh=plsc.VectorSubcoreMesh(core_axis_name="c", subcore_axis_name="s"),
        compiler_params=pltpu.CompilerParams(needs_layout_passes=False),
        scratch_types=[
            pltpu.VMEM((TABLE_PAD,), jnp.float32),
            pltpu.VMEM((TABLE_PAD,), jnp.float32),
            pltpu.VMEM((RBLK, COLS), jnp.int32),
            pltpu.VMEM((RBLK, COLS), jnp.float32),
            pltpu.VMEM((RBLK, COLS), jnp.float32),
        ],
    )
    return run(x, z, sd_pad, mu_pad)


def kernel(inputs, atomic_numbers, mean, stddev):
    z = atomic_numbers.astype(jnp.int32)
    pad = TABLE_PAD - mean.shape[0]
    sd_pad = jnp.pad(stddev.astype(jnp.float32), (0, pad))
    mu_pad = jnp.pad(mean.astype(jnp.float32), (0, pad))
    return _scale_shift(inputs, z, sd_pad, mu_pad)


# speedup vs baseline: 594.2537x; 1.0025x over previous
import jax
import jax.numpy as jnp
from jax import lax
from jax.experimental import pallas as pl
from jax.experimental.pallas import tpu as pltpu
from jax.experimental.pallas import tpu_sc as plsc

NC = 2
NS = 16
NW = NC * NS
L = 16

TABLE_PAD = 128

ROWS, COLS = 16384, 200
RPW = ROWS // NW
RBLK = 64
NBLK = RPW // RBLK
BLKE = RBLK * COLS


def _sc_body(in_hbm, idx_hbm, sd_hbm, mu_hbm, out_hbm,
             sd_v, mu_v, idx_v, in_v, out_v):
    wid = lax.axis_index("s") * NC + lax.axis_index("c")
    r0 = wid * RPW

    pltpu.sync_copy(sd_hbm, sd_v)
    pltpu.sync_copy(mu_hbm, mu_v)

    col_offs = [c * L for c in range(COLS // L)] + [COLS - L]

    def blk_body(i, _):
        rb = r0 + i * RBLK
        pltpu.sync_copy(idx_hbm.at[pl.ds(rb, RBLK), :], idx_v)
        pltpu.sync_copy(in_hbm.at[pl.ds(rb, RBLK), :], in_v)

        @plsc.parallel_loop(0, RBLK, step=1, unroll=2)
        def row_body(r):
            for c in col_offs:
                idx = idx_v[r, pl.ds(c, L)]
                sd = plsc.load_gather(sd_v, [idx])
                mu = plsc.load_gather(mu_v, [idx])
                out_v[r, pl.ds(c, L)] = in_v[r, pl.ds(c, L)] * sd + mu

        pltpu.sync_copy(out_v, out_hbm.at[pl.ds(rb, RBLK), :])
        return 0

    lax.fori_loop(0, NBLK, blk_body, 0)


@jax.jit
def _scale_shift(x, z, sd_pad, mu_pad):
    run = pl.kernel(
        _sc_body,
        out_type=jax.ShapeDtypeStruct((ROWS, COLS), jnp.float32),
        mesh=plsc.VectorSubcoreMesh(core_axis_name="c", subcore_axis_name="s"),
        compiler_params=pltpu.CompilerParams(needs_layout_passes=False,
                                             use_tc_tiling_on_sc=True),
        scratch_types=[
            pltpu.VMEM((TABLE_PAD,), jnp.float32),
            pltpu.VMEM((TABLE_PAD,), jnp.float32),
            pltpu.VMEM((RBLK, COLS), jnp.int32),
            pltpu.VMEM((RBLK, COLS), jnp.float32),
            pltpu.VMEM((RBLK, COLS), jnp.float32),
        ],
    )
    return run(x, z, sd_pad, mu_pad)


def kernel(inputs, atomic_numbers, mean, stddev):
    z = atomic_numbers.astype(jnp.int32)
    pad = TABLE_PAD - mean.shape[0]
    sd_pad = jnp.pad(stddev.astype(jnp.float32), (0, pad))
    mu_pad = jnp.pad(mean.astype(jnp.float32), (0, pad))
    return _scale_shift(inputs, z, sd_pad, mu_pad)


# speedup vs baseline: 1137.1951x; 1.9137x over previous
import jax
import jax.numpy as jnp
from jax import lax
from jax.experimental import pallas as pl
from jax.experimental.pallas import tpu as pltpu
from jax.experimental.pallas import tpu_sc as plsc

NC = 2
NS = 16
NW = NC * NS
L = 16

TABLE_PAD = 128

R, C = 200, 16384
CPW = C // NW
CBLK = 128
NBLK = CPW // CBLK


def _sc_body(in_hbm, idx_hbm, sd_hbm, mu_hbm, out_hbm,
             sd_v, mu_v, idx_v, in_v, out_v):
    wid = lax.axis_index("s") * NC + lax.axis_index("c")
    c0 = wid * CPW

    pltpu.sync_copy(sd_hbm, sd_v)
    pltpu.sync_copy(mu_hbm, mu_v)

    def blk_body(i, _):
        cb = c0 + i * CBLK
        pltpu.sync_copy(idx_hbm.at[:, pl.ds(cb, CBLK)], idx_v)
        pltpu.sync_copy(in_hbm.at[:, pl.ds(cb, CBLK)], in_v)

        @plsc.parallel_loop(0, R, step=1, unroll=2)
        def row_body(r):
            for c in range(0, CBLK, L):
                idx = idx_v[r, pl.ds(c, L)]
                sd = plsc.load_gather(sd_v, [idx])
                mu = plsc.load_gather(mu_v, [idx])
                out_v[r, pl.ds(c, L)] = in_v[r, pl.ds(c, L)] * sd + mu

        pltpu.sync_copy(out_v, out_hbm.at[:, pl.ds(cb, CBLK)])
        return 0

    lax.fori_loop(0, NBLK, blk_body, 0)


@jax.jit
def _scale_shift(xt, zt, sd_pad, mu_pad):
    run = pl.kernel(
        _sc_body,
        out_type=jax.ShapeDtypeStruct((R, C), jnp.float32),
        mesh=plsc.VectorSubcoreMesh(core_axis_name="c", subcore_axis_name="s"),
        compiler_params=pltpu.CompilerParams(needs_layout_passes=False),
        scratch_types=[
            pltpu.VMEM((TABLE_PAD,), jnp.float32),
            pltpu.VMEM((TABLE_PAD,), jnp.float32),
            pltpu.VMEM((R, CBLK), jnp.int32),
            pltpu.VMEM((R, CBLK), jnp.float32),
            pltpu.VMEM((R, CBLK), jnp.float32),
        ],
    )
    return run(xt, zt, sd_pad, mu_pad)


def kernel(inputs, atomic_numbers, mean, stddev):
    zt = atomic_numbers.astype(jnp.int32).T
    pad = TABLE_PAD - mean.shape[0]
    sd_pad = jnp.pad(stddev.astype(jnp.float32), (0, pad))
    mu_pad = jnp.pad(mean.astype(jnp.float32), (0, pad))
    yt = _scale_shift(inputs.T, zt, sd_pad, mu_pad)
    return yt.T


# speedup vs baseline: 1153.2829x; 1.0141x over previous
import jax
import jax.numpy as jnp
from jax import lax
from jax.experimental import pallas as pl
from jax.experimental.pallas import tpu as pltpu
from jax.experimental.pallas import tpu_sc as plsc

NC = 2
NS = 16
NW = NC * NS
L = 16

TABLE_PAD = 128

R, C = 200, 16384
CPW = C // NW
CBLK = 128
RBLK = 40
NRB = R // RBLK
NCB = CPW // CBLK
NB = NRB * NCB


def _sc_body(in_hbm, idx_hbm, sd_hbm, mu_hbm, out_hbm, sd_v, mu_v,
             idx0, idx1, in0, in1, out0, out1,
             si0, si1, sx0, sx1, so0, so1):
    wid = lax.axis_index("s") * NC + lax.axis_index("c")
    c0 = wid * CPW

    pltpu.sync_copy(sd_hbm, sd_v)
    pltpu.sync_copy(mu_hbm, mu_v)

    idxb, inb, outb = [idx0, idx1], [in0, in1], [out0, out1]
    si, sx, so = [si0, si1], [sx0, sx1], [so0, so1]

    def offs(i):
        j, k = divmod(i, NRB)
        return k * RBLK, c0 + j * CBLK

    def start_in(i):
        b = i % 2
        r, c = offs(i)
        hi = pltpu.async_copy(
            idx_hbm.at[pl.ds(r, RBLK), pl.ds(c, CBLK)], idxb[b], si[b])
        hx = pltpu.async_copy(
            in_hbm.at[pl.ds(r, RBLK), pl.ds(c, CBLK)], inb[b], sx[b])
        return hi, hx

    def start_out(i):
        b = i % 2
        r, c = offs(i)
        return pltpu.async_copy(
            outb[b], out_hbm.at[pl.ds(r, RBLK), pl.ds(c, CBLK)], so[b])

    hin, hout = {}, {}
    hin[0] = start_in(0)
    hin[1] = start_in(1)
    for i in range(NB):
        b = i % 2
        for h in hin.pop(i):
            h.wait()
        if i >= 2:
            hout.pop(i - 2).wait()

        @plsc.parallel_loop(0, RBLK, step=1, unroll=2)
        def row_body(r, _b=b):
            for c in range(0, CBLK, L):
                idx = idxb[_b][r, pl.ds(c, L)]
                sd = plsc.load_gather(sd_v, [idx])
                mu = plsc.load_gather(mu_v, [idx])
                outb[_b][r, pl.ds(c, L)] = inb[_b][r, pl.ds(c, L)] * sd + mu

        hout[i] = start_out(i)
        if i + 2 < NB:
            hin[i + 2] = start_in(i + 2)
    for h in hout.values():
        h.wait()


@jax.jit
def _scale_shift(xt, zt, sd_pad, mu_pad):
    run = pl.kernel(
        _sc_body,
        out_type=jax.ShapeDtypeStruct((R, C), jnp.float32),
        mesh=plsc.VectorSubcoreMesh(core_axis_name="c", subcore_axis_name="s"),
        compiler_params=pltpu.CompilerParams(needs_layout_passes=False),
        scratch_types=[
            pltpu.VMEM((TABLE_PAD,), jnp.float32),
            pltpu.VMEM((TABLE_PAD,), jnp.float32),
            pltpu.VMEM((RBLK, CBLK), jnp.int32),
            pltpu.VMEM((RBLK, CBLK), jnp.int32),
            pltpu.VMEM((RBLK, CBLK), jnp.float32),
            pltpu.VMEM((RBLK, CBLK), jnp.float32),
            pltpu.VMEM((RBLK, CBLK), jnp.float32),
            pltpu.VMEM((RBLK, CBLK), jnp.float32),
            pltpu.SemaphoreType.DMA,
            pltpu.SemaphoreType.DMA,
            pltpu.SemaphoreType.DMA,
            pltpu.SemaphoreType.DMA,
            pltpu.SemaphoreType.DMA,
            pltpu.SemaphoreType.DMA,
        ],
    )
    return run(xt, zt, sd_pad, mu_pad)


def kernel(inputs, atomic_numbers, mean, stddev):
    zt = atomic_numbers.astype(jnp.int32).T
    pad = TABLE_PAD - mean.shape[0]
    sd_pad = jnp.pad(stddev.astype(jnp.float32), (0, pad))
    mu_pad = jnp.pad(mean.astype(jnp.float32), (0, pad))
    yt = _scale_shift(inputs.T, zt, sd_pad, mu_pad)
    return yt.T


# speedup vs baseline: 1485.5774x; 1.2881x over previous
import jax
import jax.numpy as jnp
from jax import lax
from jax.experimental import pallas as pl
from jax.experimental.pallas import tpu as pltpu
from jax.experimental.pallas import tpu_sc as plsc

NC = 2
NS = 16
NW = NC * NS
L = 16

MAXZ = 100
TABLE_PAD = 128

R, C = 200, 16384
CPW = C // NW
CBLK = 128
RBLK = 40
NRB = R // RBLK
NCB = CPW // CBLK
NB = NRB * NCB


def _sc_body(in_hbm, idx_hbm, sd_hbm, mu_hbm, out_hbm, sd_v, mu_v,
             idx0, idx1, in0, in1, out0, out1,
             si0, si1, sx0, sx1, so0, so1):
    wid = lax.axis_index("s") * NC + lax.axis_index("c")
    c0 = wid * CPW

    pltpu.sync_copy(sd_hbm, sd_v.at[pl.ds(0, MAXZ)])
    pltpu.sync_copy(mu_hbm, mu_v.at[pl.ds(0, MAXZ)])

    idxb, inb, outb = [idx0, idx1], [in0, in1], [out0, out1]
    si, sx, so = [si0, si1], [sx0, sx1], [so0, so1]

    def offs(i):
        j = i // NRB
        k = i % NRB
        return k * RBLK, c0 + j * CBLK

    def start_in(i, b):
        r, c = offs(i)
        pltpu.async_copy(
            idx_hbm.at[pl.ds(r, RBLK), pl.ds(c, CBLK)], idxb[b], si[b])
        pltpu.async_copy(
            in_hbm.at[pl.ds(r, RBLK), pl.ds(c, CBLK)], inb[b], sx[b])

    def wait_in(b):
        pltpu.make_async_copy(
            idx_hbm.at[pl.ds(0, RBLK), pl.ds(0, CBLK)], idxb[b], si[b]).wait()
        pltpu.make_async_copy(
            in_hbm.at[pl.ds(0, RBLK), pl.ds(0, CBLK)], inb[b], sx[b]).wait()

    def start_out(i, b):
        r, c = offs(i)
        pltpu.async_copy(
            outb[b], out_hbm.at[pl.ds(r, RBLK), pl.ds(c, CBLK)], so[b])

    def wait_out(b):
        pltpu.make_async_copy(
            outb[b], out_hbm.at[pl.ds(0, RBLK), pl.ds(0, CBLK)], so[b]).wait()

    def compute(b):
        @plsc.parallel_loop(0, RBLK, step=1, unroll=4)
        def row_body(r):
            for c in range(0, CBLK, L):
                idx = idxb[b][r, pl.ds(c, L)]
                sd = plsc.load_gather(sd_v, [idx])
                mu = plsc.load_gather(mu_v, [idx])
                outb[b][r, pl.ds(c, L)] = inb[b][r, pl.ds(c, L)] * sd + mu

    start_in(0, 0)
    start_in(1, 1)

    @pl.loop(0, NB, step=2)
    def ring(g):
        for b in (0, 1):
            i = g + b
            wait_in(b)

            @pl.when(i >= 2)
            def _():
                wait_out(b)

            compute(b)
            start_out(i, b)

            @pl.when(i + 2 < NB)
            def _():
                start_in(i + 2, b)

    wait_out(0)
    wait_out(1)


@jax.jit
def _scale_shift(xt, zt, sd, mu):
    run = pl.kernel(
        _sc_body,
        out_type=jax.ShapeDtypeStruct((R, C), jnp.float32),
        mesh=plsc.VectorSubcoreMesh(core_axis_name="c", subcore_axis_name="s"),
        compiler_params=pltpu.CompilerParams(needs_layout_passes=False),
        scratch_types=[
            pltpu.VMEM((TABLE_PAD,), jnp.float32),
            pltpu.VMEM((TABLE_PAD,), jnp.float32),
            pltpu.VMEM((RBLK, CBLK), jnp.int32),
            pltpu.VMEM((RBLK, CBLK), jnp.int32),
            pltpu.VMEM((RBLK, CBLK), jnp.float32),
            pltpu.VMEM((RBLK, CBLK), jnp.float32),
            pltpu.VMEM((RBLK, CBLK), jnp.float32),
            pltpu.VMEM((RBLK, CBLK), jnp.float32),
            pltpu.SemaphoreType.DMA,
            pltpu.SemaphoreType.DMA,
            pltpu.SemaphoreType.DMA,
            pltpu.SemaphoreType.DMA,
            pltpu.SemaphoreType.DMA,
            pltpu.SemaphoreType.DMA,
        ],
    )
    return run(xt, zt, sd, mu)


def kernel(inputs, atomic_numbers, mean, stddev):
    zt = atomic_numbers.astype(jnp.int32).T
    yt = _scale_shift(inputs.T, zt,
                      stddev.astype(jnp.float32), mean.astype(jnp.float32))
    return yt.T
